# Initial kernel scaffold; baseline (speedup 1.0000x reference)
import jax
import jax.numpy as jnp
from jax.experimental import pallas as pl


def kernel(indices, emb_table):
    raise NotImplementedError("write your pallas kernel here")


# speedup vs baseline: 1.2609x; 1.2609x over previous
import functools
import math

import jax
import jax.numpy as jnp
from jax.experimental import pallas as pl
from jax.experimental.pallas import tpu as pltpu


def _ceil_to(x, m):
    return (x + m - 1) // m * m


_TOKENS_PER_TILE = 128


def _vld_gather_kernel(idx_ref, emb_ref, out_ref, *, tm, scale):
    base = pl.program_id(0) * tm
    for mi in range(tm):
        row = emb_ref[idx_ref[base + mi], 0]
        out_ref[mi, 0] = row * scale


def kernel(indices, emb_table):
    b, s = indices.shape
    v, d = emb_table.shape
    n = b * s
    scale = math.sqrt(float(d))

    flat_idx = jnp.clip(indices.reshape(n).astype(jnp.int32), 0, v - 1)

    tm = min(_TOKENS_PER_TILE, n)
    n_pad = _ceil_to(n, tm)
    if n_pad != n:
        flat_idx = jnp.pad(flat_idx, (0, n_pad - n))
    num_tiles = n_pad // tm

    emb3 = emb_table.reshape(v, 1, d)

    table_bytes = v * d * emb_table.dtype.itemsize
    out_tile_bytes = tm * d * emb_table.dtype.itemsize
    vmem_limit = min(table_bytes + 4 * out_tile_bytes + (4 << 20), 60 << 20)

    out = pl.pallas_call(
        functools.partial(_vld_gather_kernel, tm=tm, scale=scale),
        out_shape=jax.ShapeDtypeStruct((n_pad, 1, d), emb_table.dtype),
        grid_spec=pltpu.PrefetchScalarGridSpec(
            num_scalar_prefetch=1,
            grid=(num_tiles,),
            in_specs=[pl.BlockSpec(memory_space=pltpu.MemorySpace.VMEM)],
            out_specs=pl.BlockSpec((tm, 1, d), lambda i, idx_ref: (i, 0, 0)),
        ),
        compiler_params=pltpu.CompilerParams(
            dimension_semantics=("parallel",),
            vmem_limit_bytes=int(vmem_limit),
        ),
    )(flat_idx, emb3)

    return out.reshape(n_pad, d)[:n].reshape(b, s, d)


# speedup vs baseline: 1.5656x; 1.2416x over previous
import functools
import math

import jax
import jax.numpy as jnp
from jax.experimental import pallas as pl
from jax.experimental.pallas import tpu as pltpu


def _ceil_to(x, m):
    return (x + m - 1) // m * m


_TOKENS_PER_TILE = 128


def _vld_gather_kernel(idx_ref, emb_ref, out_ref, *, tm, scale):
    base = pl.program_id(0) * tm
    for mi in range(tm):
        row = emb_ref[idx_ref[base + mi], 0]
        out_ref[mi, :] = row * scale


def kernel(indices, emb_table):
    b, s = indices.shape
    v, d = emb_table.shape
    n = b * s
    scale = math.sqrt(float(d))

    flat_idx = jnp.clip(indices.reshape(n).astype(jnp.int32), 0, v - 1)

    tm = min(_TOKENS_PER_TILE, n)
    n_pad = _ceil_to(n, tm)
    if n_pad != n:
        flat_idx = jnp.pad(flat_idx, (0, n_pad - n))
    num_tiles = n_pad // tm

    emb3 = emb_table.reshape(v, 1, d)

    table_bytes = v * d * emb_table.dtype.itemsize
    out_tile_bytes = tm * d * emb_table.dtype.itemsize
    vmem_limit = min(table_bytes + 4 * out_tile_bytes + (4 << 20), 60 << 20)

    out = pl.pallas_call(
        functools.partial(_vld_gather_kernel, tm=tm, scale=scale),
        out_shape=jax.ShapeDtypeStruct((n_pad, d), emb_table.dtype),
        grid_spec=pltpu.PrefetchScalarGridSpec(
            num_scalar_prefetch=1,
            grid=(num_tiles,),
            in_specs=[pl.BlockSpec(memory_space=pltpu.MemorySpace.VMEM)],
            out_specs=pl.BlockSpec((tm, d), lambda i, idx_ref: (i, 0)),
        ),
        compiler_params=pltpu.CompilerParams(
            dimension_semantics=("parallel",),
            vmem_limit_bytes=int(vmem_limit),
        ),
    )(flat_idx, emb3)

    return out[:n].reshape(b, s, d)


# speedup vs baseline: 2.4283x; 1.5510x over previous
import functools
import math

import jax
import jax.numpy as jnp
from jax.experimental import pallas as pl
from jax.experimental.pallas import tpu as pltpu


def _ceil_to(x, m):
    return (x + m - 1) // m * m


_ROWS_PER_STEP = 256


def _dma_gather_kernel(idx_ref, emb_hbm, out_ref, sem, *, rows, scale):
    step = pl.program_id(0) * pl.num_programs(1) + pl.program_id(1)
    base = step * rows
    for r in range(rows):
        pltpu.make_async_copy(
            emb_hbm.at[pl.ds(idx_ref[base + r], 1), :],
            out_ref.at[pl.ds(r, 1), :],
            sem,
        ).start()
    pltpu.make_async_copy(
        emb_hbm.at[pl.ds(0, rows), :],
        out_ref.at[pl.ds(0, rows), :],
        sem,
    ).wait()
    out_ref[...] = out_ref[...] * scale


def kernel(indices, emb_table):
    b, s = indices.shape
    v, d = emb_table.shape
    n = b * s
    scale = math.sqrt(float(d))

    flat_idx = jnp.clip(indices.reshape(n).astype(jnp.int32), 0, v - 1)

    rows = min(_ROWS_PER_STEP, n)
    n_pad = _ceil_to(n, 2 * rows)
    if n_pad != n:
        flat_idx = jnp.pad(flat_idx, (0, n_pad - n))
    n_steps = n_pad // (2 * rows)

    out = pl.pallas_call(
        functools.partial(_dma_gather_kernel, rows=rows, scale=scale),
        out_shape=jax.ShapeDtypeStruct((n_pad, d), emb_table.dtype),
        grid_spec=pltpu.PrefetchScalarGridSpec(
            num_scalar_prefetch=1,
            grid=(2, n_steps),
            in_specs=[pl.BlockSpec(memory_space=pl.ANY)],
            out_specs=pl.BlockSpec(
                (rows, d), lambda c, i, idx_ref: (c * pl.num_programs(1) + i, 0)
            ),
            scratch_shapes=[pltpu.SemaphoreType.DMA],
        ),
        compiler_params=pltpu.CompilerParams(
            dimension_semantics=("parallel", "arbitrary"),
            vmem_limit_bytes=32 << 20,
        ),
    )(flat_idx, emb_table)

    return out[:n].reshape(b, s, d)


# speedup vs baseline: 2.5841x; 1.0641x over previous
import functools
import math

import jax
import jax.numpy as jnp
from jax.experimental import pallas as pl
from jax.experimental.pallas import tpu as pltpu


def _ceil_to(x, m):
    return (x + m - 1) // m * m


_ROWS_PER_STEP = 512
_WAVE = 128


def _dma_gather_kernel(idx_ref, emb_hbm, out_ref, sems, *, rows, wave, scale):
    step = pl.program_id(0) * pl.num_programs(1) + pl.program_id(1)
    base = step * rows
    n_waves = rows // wave

    def issue(w):
        sem = sems.at[w % 2]
        for r in range(wave):
            pltpu.make_async_copy(
                emb_hbm.at[pl.ds(idx_ref[base + w * wave + r], 1), :],
                out_ref.at[pl.ds(w * wave + r, 1), :],
                sem,
            ).start()

    def drain(w):
        pltpu.make_async_copy(
            emb_hbm.at[pl.ds(0, wave), :],
            out_ref.at[pl.ds(w * wave, wave), :],
            sems.at[w % 2],
        ).wait()
        off = w * wave
        out_ref[pl.ds(off, wave), :] = out_ref[pl.ds(off, wave), :] * scale

    for w in range(n_waves):
        issue(w)
        if w >= 1:
            drain(w - 1)
    drain(n_waves - 1)


def kernel(indices, emb_table):
    b, s = indices.shape
    v, d = emb_table.shape
    n = b * s
    scale = math.sqrt(float(d))

    flat_idx = jnp.clip(indices.reshape(n).astype(jnp.int32), 0, v - 1)

    rows = min(_ROWS_PER_STEP, n)
    wave = min(_WAVE, rows)
    n_pad = _ceil_to(n, 2 * rows)
    if n_pad != n:
        flat_idx = jnp.pad(flat_idx, (0, n_pad - n))
    n_steps = n_pad // (2 * rows)

    out = pl.pallas_call(
        functools.partial(_dma_gather_kernel, rows=rows, wave=wave, scale=scale),
        out_shape=jax.ShapeDtypeStruct((n_pad, d), emb_table.dtype),
        grid_spec=pltpu.PrefetchScalarGridSpec(
            num_scalar_prefetch=1,
            grid=(2, n_steps),
            in_specs=[pl.BlockSpec(memory_space=pl.ANY)],
            out_specs=pl.BlockSpec(
                (rows, d), lambda c, i, idx_ref: (c * pl.num_programs(1) + i, 0)
            ),
            scratch_shapes=[pltpu.SemaphoreType.DMA((2,))],
        ),
        compiler_params=pltpu.CompilerParams(
            dimension_semantics=("parallel", "arbitrary"),
            vmem_limit_bytes=32 << 20,
        ),
    )(flat_idx, emb_table)

    return out[:n].reshape(b, s, d)


# speedup vs baseline: 2.7979x; 1.0827x over previous
import functools
import math

import jax
import jax.numpy as jnp
from jax.experimental import pallas as pl
from jax.experimental.pallas import tpu as pltpu


def _ceil_to(x, m):
    return (x + m - 1) // m * m


_WAVE = 128


def _dma_gather_kernel(idx_ref, emb_hbm, out_hbm, buf, rsems, wsem,
                       *, tokens_per_core, wave, scale):
    core = pl.program_id(0)
    tbase = core * tokens_per_core
    n_waves = tokens_per_core // wave

    def issue_read(w):
        sem = rsems.at[w % 2]
        for r in range(wave):
            tok = w * wave + r
            pltpu.make_async_copy(
                emb_hbm.at[pl.ds(idx_ref[tbase + tok], 1), :],
                buf.at[pl.ds(tok, 1), :],
                sem,
            ).start()

    def drain_and_write(w):
        off = w * wave
        pltpu.make_async_copy(
            emb_hbm.at[pl.ds(0, wave), :],
            buf.at[pl.ds(off, wave), :],
            rsems.at[w % 2],
        ).wait()
        buf[pl.ds(off, wave), :] = buf[pl.ds(off, wave), :] * scale
        pltpu.make_async_copy(
            buf.at[pl.ds(off, wave), :],
            out_hbm.at[pl.ds(tbase + off, wave), :],
            wsem,
        ).start()

    issue_read(0)
    for w in range(n_waves):
        if w + 1 < n_waves:
            issue_read(w + 1)
        drain_and_write(w)
    pltpu.make_async_copy(
        buf.at[pl.ds(0, tokens_per_core), :],
        out_hbm.at[pl.ds(tbase, tokens_per_core), :],
        wsem,
    ).wait()


def kernel(indices, emb_table):
    b, s = indices.shape
    v, d = emb_table.shape
    n = b * s
    scale = math.sqrt(float(d))

    flat_idx = jnp.clip(indices.reshape(n).astype(jnp.int32), 0, v - 1)

    wave = min(_WAVE, n)
    n_pad = _ceil_to(n, 2 * wave)
    if n_pad != n:
        flat_idx = jnp.pad(flat_idx, (0, n_pad - n))
    tokens_per_core = n_pad // 2

    out = pl.pallas_call(
        functools.partial(_dma_gather_kernel, tokens_per_core=tokens_per_core,
                          wave=wave, scale=scale),
        out_shape=jax.ShapeDtypeStruct((n_pad, d), emb_table.dtype),
        grid_spec=pltpu.PrefetchScalarGridSpec(
            num_scalar_prefetch=1,
            grid=(2,),
            in_specs=[pl.BlockSpec(memory_space=pl.ANY)],
            out_specs=pl.BlockSpec(memory_space=pl.ANY),
            scratch_shapes=[
                pltpu.VMEM((tokens_per_core, d), emb_table.dtype),
                pltpu.SemaphoreType.DMA((2,)),
                pltpu.SemaphoreType.DMA,
            ],
        ),
        compiler_params=pltpu.CompilerParams(
            dimension_semantics=("parallel",),
            vmem_limit_bytes=40 << 20,
        ),
    )(flat_idx, emb_table)

    return out[:n].reshape(b, s, d)


# speedup vs baseline: 2.8433x; 1.0162x over previous
import functools
import math

import jax
import jax.numpy as jnp
from jax.experimental import pallas as pl
from jax.experimental.pallas import tpu as pltpu


def _ceil_to(x, m):
    return (x + m - 1) // m * m


_WAVE = 128


def _dma_gather_kernel(idx_ref, emb_hbm, out_hbm, buf, rsems, wsem,
                       *, tokens_per_core, wave, scale):
    core = pl.program_id(0)
    tbase = core * tokens_per_core
    n_waves = tokens_per_core // wave

    def issue_read(w):
        sem = rsems.at[w % 2]
        for r in range(wave):
            tok = w * wave + r
            pltpu.make_async_copy(
                emb_hbm.at[pl.ds(idx_ref[tbase + tok], 1), :],
                buf.at[pl.ds(tok, 1), :],
                sem,
            ).start(priority=r % 2)

    def drain_and_write(w):
        off = w * wave
        pltpu.make_async_copy(
            emb_hbm.at[pl.ds(0, wave), :],
            buf.at[pl.ds(off, wave), :],
            rsems.at[w % 2],
        ).wait()
        buf[pl.ds(off, wave), :] = buf[pl.ds(off, wave), :] * scale
        pltpu.make_async_copy(
            buf.at[pl.ds(off, wave), :],
            out_hbm.at[pl.ds(tbase + off, wave), :],
            wsem,
        ).start()

    issue_read(0)
    for w in range(n_waves):
        if w + 1 < n_waves:
            issue_read(w + 1)
        drain_and_write(w)
    pltpu.make_async_copy(
        buf.at[pl.ds(0, tokens_per_core), :],
        out_hbm.at[pl.ds(tbase, tokens_per_core), :],
        wsem,
    ).wait()


def kernel(indices, emb_table):
    b, s = indices.shape
    v, d = emb_table.shape
    n = b * s
    scale = math.sqrt(float(d))

    flat_idx = jnp.clip(indices.reshape(n).astype(jnp.int32), 0, v - 1)

    wave = min(_WAVE, n)
    n_pad = _ceil_to(n, 2 * wave)
    if n_pad != n:
        flat_idx = jnp.pad(flat_idx, (0, n_pad - n))
    tokens_per_core = n_pad // 2

    out = pl.pallas_call(
        functools.partial(_dma_gather_kernel, tokens_per_core=tokens_per_core,
                          wave=wave, scale=scale),
        out_shape=jax.ShapeDtypeStruct((n_pad, d), emb_table.dtype),
        grid_spec=pltpu.PrefetchScalarGridSpec(
            num_scalar_prefetch=1,
            grid=(2,),
            in_specs=[pl.BlockSpec(memory_space=pl.ANY)],
            out_specs=pl.BlockSpec(memory_space=pl.ANY),
            scratch_shapes=[
                pltpu.VMEM((tokens_per_core, d), emb_table.dtype),
                pltpu.SemaphoreType.DMA((2,)),
                pltpu.SemaphoreType.DMA,
            ],
        ),
        compiler_params=pltpu.CompilerParams(
            dimension_semantics=("parallel",),
            vmem_limit_bytes=40 << 20,
        ),
    )(flat_idx, emb_table)

    return out[:n].reshape(b, s, d)


# speedup vs baseline: 3.9661x; 1.3949x over previous
import functools
import math

import jax
import jax.numpy as jnp
from jax.experimental import pallas as pl
from jax.experimental.pallas import tpu as pltpu


def _ceil_to(x, m):
    return (x + m - 1) // m * m


_WAVE = 256


def _dma_gather_kernel(idx_ref, emb_hbm, out_hbm, buf, rsems, wsem,
                       *, tokens_per_core, wave, scale):
    core = pl.program_id(0)
    tbase = core * tokens_per_core
    n_waves = tokens_per_core // wave

    def issue_read(w):
        sem = rsems.at[w % 2]
        for r in range(wave):
            tok = w * wave + r
            pltpu.make_async_copy(
                emb_hbm.at[pl.ds(idx_ref[tbase + tok], 1), :],
                buf.at[pl.ds(tok, 1), :],
                sem,
            ).start(priority=r % 2)

    def drain_and_write(w):
        off = w * wave
        pltpu.make_async_copy(
            emb_hbm.at[pl.ds(0, wave), :],
            buf.at[pl.ds(off, wave), :],
            rsems.at[w % 2],
        ).wait()
        buf[pl.ds(off, wave), :] = buf[pl.ds(off, wave), :] * scale
        pltpu.make_async_copy(
            buf.at[pl.ds(off, wave), :],
            out_hbm.at[pl.ds(tbase + off, wave), :],
            wsem,
        ).start()

    issue_read(0)
    for w in range(n_waves):
        if w + 1 < n_waves:
            issue_read(w + 1)
        drain_and_write(w)
    pltpu.make_async_copy(
        buf.at[pl.ds(0, tokens_per_core), :],
        out_hbm.at[pl.ds(tbase, tokens_per_core), :],
        wsem,
    ).wait()


def kernel(indices, emb_table):
    b, s = indices.shape
    v, d = emb_table.shape
    n = b * s
    scale = math.sqrt(float(d))

    flat_idx = jnp.clip(indices.reshape(n).astype(jnp.int32), 0, v - 1)

    wave = min(_WAVE, n)
    n_pad = _ceil_to(n, 2 * wave)
    if n_pad != n:
        flat_idx = jnp.pad(flat_idx, (0, n_pad - n))
    tokens_per_core = n_pad // 2

    out = pl.pallas_call(
        functools.partial(_dma_gather_kernel, tokens_per_core=tokens_per_core,
                          wave=wave, scale=scale),
        out_shape=jax.ShapeDtypeStruct((n_pad, d), emb_table.dtype),
        grid_spec=pltpu.PrefetchScalarGridSpec(
            num_scalar_prefetch=1,
            grid=(2,),
            in_specs=[pl.BlockSpec(memory_space=pl.ANY)],
            out_specs=pl.BlockSpec(memory_space=pl.ANY),
            scratch_shapes=[
                pltpu.VMEM((tokens_per_core, d), emb_table.dtype),
                pltpu.SemaphoreType.DMA((2,)),
                pltpu.SemaphoreType.DMA,
            ],
        ),
        compiler_params=pltpu.CompilerParams(
            dimension_semantics=("parallel",),
            vmem_limit_bytes=40 << 20,
        ),
    )(flat_idx, emb_table)

    return out[:n].reshape(b, s, d)


# speedup vs baseline: 4.0773x; 1.0280x over previous
import functools
import math

import jax
import jax.numpy as jnp
from jax.experimental import pallas as pl
from jax.experimental.pallas import tpu as pltpu


def _ceil_to(x, m):
    return (x + m - 1) // m * m


_WAVE = 512


def _dma_gather_kernel(idx_ref, emb_hbm, out_hbm, buf, rsems, wsem,
                       *, tokens_per_core, wave, scale):
    core = pl.program_id(0)
    tbase = core * tokens_per_core
    n_waves = tokens_per_core // wave

    def issue_read(w):
        sem = rsems.at[w % 2]
        for r in range(wave):
            tok = w * wave + r
            pltpu.make_async_copy(
                emb_hbm.at[pl.ds(idx_ref[tbase + tok], 1), :],
                buf.at[pl.ds(tok, 1), :],
                sem,
            ).start(priority=r % 2)

    def drain_and_write(w):
        off = w * wave
        pltpu.make_async_copy(
            emb_hbm.at[pl.ds(0, wave), :],
            buf.at[pl.ds(off, wave), :],
            rsems.at[w % 2],
        ).wait()
        buf[pl.ds(off, wave), :] = buf[pl.ds(off, wave), :] * scale
        pltpu.make_async_copy(
            buf.at[pl.ds(off, wave), :],
            out_hbm.at[pl.ds(tbase + off, wave), :],
            wsem,
        ).start()

    issue_read(0)
    for w in range(n_waves):
        if w + 1 < n_waves:
            issue_read(w + 1)
        drain_and_write(w)
    pltpu.make_async_copy(
        buf.at[pl.ds(0, tokens_per_core), :],
        out_hbm.at[pl.ds(tbase, tokens_per_core), :],
        wsem,
    ).wait()


def kernel(indices, emb_table):
    b, s = indices.shape
    v, d = emb_table.shape
    n = b * s
    scale = math.sqrt(float(d))

    flat_idx = jnp.clip(indices.reshape(n).astype(jnp.int32), 0, v - 1)

    wave = min(_WAVE, n)
    n_pad = _ceil_to(n, 2 * wave)
    if n_pad != n:
        flat_idx = jnp.pad(flat_idx, (0, n_pad - n))
    tokens_per_core = n_pad // 2

    out = pl.pallas_call(
        functools.partial(_dma_gather_kernel, tokens_per_core=tokens_per_core,
                          wave=wave, scale=scale),
        out_shape=jax.ShapeDtypeStruct((n_pad, d), emb_table.dtype),
        grid_spec=pltpu.PrefetchScalarGridSpec(
            num_scalar_prefetch=1,
            grid=(2,),
            in_specs=[pl.BlockSpec(memory_space=pl.ANY)],
            out_specs=pl.BlockSpec(memory_space=pl.ANY),
            scratch_shapes=[
                pltpu.VMEM((tokens_per_core, d), emb_table.dtype),
                pltpu.SemaphoreType.DMA((2,)),
                pltpu.SemaphoreType.DMA,
            ],
        ),
        compiler_params=pltpu.CompilerParams(
            dimension_semantics=("parallel",),
            vmem_limit_bytes=40 << 20,
        ),
    )(flat_idx, emb_table)

    return out[:n].reshape(b, s, d)
